# Initial kernel scaffold; baseline (speedup 1.0000x reference)
import functools

import jax
import jax.numpy as jnp
from jax import lax
from jax.experimental import pallas as pl
from jax.experimental.pallas import tpu as pltpu
from jax.experimental.pallas import tpu_sc as plsc

NC = 2
NS = 16
NW = NC * NS
NG = 64
K = 80
ZR = 125

_MESH = plsc.VectorSubcoreMesh(
    core_axis_name="c", subcore_axis_name="s", num_cores=NC, num_subcores=NS
)


def _build_deg(N, E):
  EPW = E // NW
  CH = EPW // K
  RPT = N // NS
  ZCH = RPT // ZR

  @functools.partial(
      pl.kernel,
      out_type=jax.ShapeDtypeStruct((NC, N, 16), jnp.float32),
      mesh=_MESH,
      scratch_types=[
          pltpu.VMEM((K,), jnp.int32),
          pltpu.VMEM((K, 16), jnp.float32),
          pltpu.VMEM((ZR, 16), jnp.float32),
          pltpu.VMEM_SHARED((N, 16), jnp.float32),
      ],
  )
  def deg_kernel(dst_hbm, ones_hbm, z_hbm, degp_hbm, idx_v, ones_v, zbuf_v,
                 table_sh):
    cid = lax.axis_index("c")
    sid = lax.axis_index("s")
    w = cid * NS + sid

    pltpu.sync_copy(ones_hbm, ones_v)
    pltpu.sync_copy(z_hbm, zbuf_v)
    row0 = sid * RPT
    for k in range(ZCH):
      pltpu.sync_copy(zbuf_v, table_sh.at[pl.ds(row0 + k * ZR, ZR)])
    plsc.subcore_barrier()

    def chunk(j, carry):
      base = w * EPW + j * K
      pltpu.sync_copy(dst_hbm.at[pl.ds(base, K)], idx_v)
      pltpu.sync_copy(ones_v, table_sh.at[idx_v], add=True)
      return carry

    lax.fori_loop(0, CH, chunk, 0)
    plsc.subcore_barrier()

    for k in range(ZCH):
      pltpu.sync_copy(table_sh.at[pl.ds(row0 + k * ZR, ZR)], zbuf_v)
      pltpu.sync_copy(zbuf_v, degp_hbm.at[cid, pl.ds(row0 + k * ZR, ZR)])

  return deg_kernel


def _build_prop(N, E, H):
  EPW = E // NW
  CH = EPW // K
  RPT = N // NS
  ZCH = RPT // ZR

  @functools.partial(
      pl.kernel,
      out_type=jax.ShapeDtypeStruct((NC, N, H), jnp.float32),
      mesh=_MESH,
      scratch_types=[
          pltpu.VMEM((K,), jnp.int32),
          pltpu.VMEM((K,), jnp.int32),
          pltpu.VMEM((K, H), jnp.float32),
          pltpu.VMEM((ZR, H), jnp.float32),
          pltpu.VMEM_SHARED((N, H), jnp.float32),
          pltpu.SemaphoreType.DMA,
      ],
  )
  def prop_kernel(hs_hbm, src_hbm, dst_hbm, z_hbm, parts_hbm, sidx_v, didx_v,
                  rows_v, zbuf_v, acc_sh, sem):
    cid = lax.axis_index("c")
    sid = lax.axis_index("s")
    w = cid * NS + sid

    pltpu.sync_copy(z_hbm, zbuf_v)
    row0 = sid * RPT
    for k in range(ZCH):
      pltpu.sync_copy(zbuf_v, acc_sh.at[pl.ds(row0 + k * ZR, ZR)])
    plsc.subcore_barrier()

    def chunk(j, carry):
      base = w * EPW + j * K
      pltpu.sync_copy(src_hbm.at[pl.ds(base, K)], sidx_v)
      pltpu.sync_copy(dst_hbm.at[pl.ds(base, K)], didx_v)
      pltpu.async_copy(hs_hbm.at[sidx_v], rows_v, sem).wait()
      pltpu.sync_copy(rows_v, acc_sh.at[didx_v], add=True)
      return carry

    lax.fori_loop(0, CH, chunk, 0)
    plsc.subcore_barrier()

    for k in range(ZCH):
      pltpu.sync_copy(acc_sh.at[pl.ds(row0 + k * ZR, ZR)], zbuf_v)
      pltpu.sync_copy(zbuf_v, parts_hbm.at[cid, pl.ds(row0 + k * ZR, ZR)])

  return prop_kernel


def _build_pool(Np, H):
  RPW = Np // NW
  CR = 80
  NCH = RPW // CR
  F3 = 3 * H

  @functools.partial(
      pl.kernel,
      out_type=jax.ShapeDtypeStruct((NW, NG, F3), jnp.float32),
      mesh=_MESH,
      scratch_types=[
          pltpu.VMEM((CR,), jnp.int32),
          pltpu.VMEM((CR, H), jnp.float32),
          pltpu.VMEM((CR, H), jnp.float32),
          pltpu.VMEM((CR, H), jnp.float32),
          pltpu.VMEM((NG, F3), jnp.float32),
      ],
  )
  def pool_kernel(x1_hbm, x2_hbm, x3_hbm, batch_hbm, neginf_hbm, out_hbm,
                  bidx_v, xa_v, xb_v, xc_v, pbuf_v):
    cid = lax.axis_index("c")
    sid = lax.axis_index("s")
    w = cid * NS + sid

    pltpu.sync_copy(neginf_hbm, pbuf_v)

    def chunkfn(t, carry):
      base = w * RPW + t * CR
      pltpu.sync_copy(batch_hbm.at[pl.ds(base, CR)], bidx_v)
      pltpu.sync_copy(x1_hbm.at[pl.ds(base, CR)], xa_v)
      pltpu.sync_copy(x2_hbm.at[pl.ds(base, CR)], xb_v)
      pltpu.sync_copy(x3_hbm.at[pl.ds(base, CR)], xc_v)

      def rowfn(r, c2):
        g = bidx_v[r]
        for a, buf in enumerate((xa_v, xb_v, xc_v)):
          for k in range(H // 16):
            v = buf[r, pl.ds(k * 16, 16)]
            cur = pbuf_v[g, pl.ds(a * H + k * 16, 16)]
            pbuf_v[g, pl.ds(a * H + k * 16, 16)] = jnp.maximum(cur, v)
        return c2

      lax.fori_loop(0, CR, rowfn, 0)
      return carry

    lax.fori_loop(0, NCH, chunkfn, 0)
    pltpu.sync_copy(pbuf_v, out_hbm.at[w])

  return pool_kernel


def _tc_first(degp, x, W1):
  N, Fin = x.shape
  H = W1.shape[1]
  B = 1000

  def body(degp_ref, x_ref, w_ref, dinv_ref, hs_ref):
    deg = 1.0 + degp_ref[0, :, 0:1] + degp_ref[1, :, 0:1]
    dinv = 1.0 / jnp.sqrt(deg)
    dinv_ref[...] = dinv
    hs_ref[...] = jnp.dot(
        x_ref[...], w_ref[...], preferred_element_type=jnp.float32) * dinv

  return pl.pallas_call(
      body,
      grid=(N // B,),
      in_specs=[
          pl.BlockSpec((NC, B, 16), lambda i: (0, i, 0)),
          pl.BlockSpec((B, Fin), lambda i: (i, 0)),
          pl.BlockSpec((Fin, H), lambda i: (0, 0)),
      ],
      out_specs=[
          pl.BlockSpec((B, 1), lambda i: (i, 0)),
          pl.BlockSpec((B, H), lambda i: (i, 0)),
      ],
      out_shape=[
          jax.ShapeDtypeStruct((N, 1), jnp.float32),
          jax.ShapeDtypeStruct((N, H), jnp.float32),
      ],
  )(degp, x, W1)


def _tc_combine(parts, hs, dinv, b_row, Wn):
  N, H = hs.shape
  B = 1000

  def body(p_ref, hs_ref, dinv_ref, b_ref, w_ref, y_ref, hsn_ref):
    y = dinv_ref[...] * (p_ref[0] + p_ref[1] + hs_ref[...]) + b_ref[...]
    y = jnp.maximum(y, 0.0)
    y_ref[...] = y
    hsn_ref[...] = jnp.dot(
        y, w_ref[...], preferred_element_type=jnp.float32) * dinv_ref[...]

  return pl.pallas_call(
      body,
      grid=(N // B,),
      in_specs=[
          pl.BlockSpec((NC, B, H), lambda i: (0, i, 0)),
          pl.BlockSpec((B, H), lambda i: (i, 0)),
          pl.BlockSpec((B, 1), lambda i: (i, 0)),
          pl.BlockSpec((1, H), lambda i: (0, 0)),
          pl.BlockSpec((H, H), lambda i: (0, 0)),
      ],
      out_specs=[
          pl.BlockSpec((B, H), lambda i: (i, 0)),
          pl.BlockSpec((B, H), lambda i: (i, 0)),
      ],
      out_shape=[
          jax.ShapeDtypeStruct((N, H), jnp.float32),
          jax.ShapeDtypeStruct((N, H), jnp.float32),
      ],
  )(parts, hs, dinv, b_row, Wn)


def _tc_combine_last(parts, hs, dinv, b_row):
  N, H = hs.shape
  B = 1000

  def body(p_ref, hs_ref, dinv_ref, b_ref, y_ref):
    y = dinv_ref[...] * (p_ref[0] + p_ref[1] + hs_ref[...]) + b_ref[...]
    y_ref[...] = jnp.maximum(y, 0.0)

  return pl.pallas_call(
      body,
      grid=(N // B,),
      in_specs=[
          pl.BlockSpec((NC, B, H), lambda i: (0, i, 0)),
          pl.BlockSpec((B, H), lambda i: (i, 0)),
          pl.BlockSpec((B, 1), lambda i: (i, 0)),
          pl.BlockSpec((1, H), lambda i: (0, 0)),
      ],
      out_specs=pl.BlockSpec((B, H), lambda i: (i, 0)),
      out_shape=jax.ShapeDtypeStruct((N, H), jnp.float32),
  )(parts, hs, dinv, b_row)


def _tc_head(pp, Wl1, bl1_row, Wl2, bl2_row):
  NGg = pp.shape[1]
  C = Wl2.shape[1]

  def body(pp_ref, w1_ref, b1_ref, w2_ref, b2_ref, o_ref):
    pooled = jnp.max(pp_ref[...], axis=0)
    h = jnp.dot(pooled, w1_ref[...], preferred_element_type=jnp.float32)
    h = jnp.maximum(h + b1_ref[...], 0.0)
    logits = jnp.dot(h, w2_ref[...], preferred_element_type=jnp.float32)
    logits = logits + b2_ref[...]
    m = jnp.max(logits, axis=-1, keepdims=True)
    lse = m + jnp.log(jnp.sum(jnp.exp(logits - m), axis=-1, keepdims=True))
    o_ref[...] = logits - lse

  return pl.pallas_call(
      body,
      out_shape=jax.ShapeDtypeStruct((NGg, C), jnp.float32),
  )(pp, Wl1, bl1_row, Wl2, bl2_row)


def kernel(x, edge_index, batch, W1, b1, W2, b2, W3, b3, Wl1, bl1, Wl2, bl2):
  N, _ = x.shape
  H = W1.shape[1]
  E = edge_index.shape[1]
  C = Wl2.shape[1]
  src = edge_index[0]
  dst = edge_index[1]

  deg_k = _build_deg(N, E)
  prop_k = _build_prop(N, E, H)

  ones16 = jnp.ones((K, 16), jnp.float32)
  z16 = jnp.zeros((ZR, 16), jnp.float32)
  zH = jnp.zeros((ZR, H), jnp.float32)

  degp = deg_k(dst, ones16, z16)
  dinv, hs1 = _tc_first(degp, x, W1)

  p = prop_k(hs1, src, dst, zH)
  x1, hs2 = _tc_combine(p, hs1, dinv, b1.reshape(1, H), W2)
  p = prop_k(hs2, src, dst, zH)
  x2, hs3 = _tc_combine(p, hs2, dinv, b2.reshape(1, H), W3)
  p = prop_k(hs3, src, dst, zH)
  x3 = _tc_combine_last(p, hs3, dinv, b3.reshape(1, H))

  RPW = -(-N // (NW * 80)) * 80
  Np = NW * RPW
  if Np != N:
    pad = jnp.full((Np - N, H), -jnp.inf, jnp.float32)
    x1p = jnp.concatenate([x1, pad], axis=0)
    x2p = jnp.concatenate([x2, pad], axis=0)
    x3p = jnp.concatenate([x3, pad], axis=0)
    batch_p = jnp.concatenate(
        [batch, jnp.full((Np - N,), NG - 1, batch.dtype)])
  else:
    x1p, x2p, x3p, batch_p = x1, x2, x3, batch

  pool_k = _build_pool(Np, H)
  neginf = jnp.full((NG, 3 * H), -jnp.inf, jnp.float32)
  pp = pool_k(x1p, x2p, x3p, batch_p, neginf)

  return _tc_head(pp, Wl1, bl1.reshape(1, H), Wl2, bl2.reshape(1, C))


# speedup vs baseline: 6.8546x; 6.8546x over previous
import functools

import jax
import jax.numpy as jnp
from jax import lax
from jax.experimental import pallas as pl
from jax.experimental.pallas import tpu as pltpu
from jax.experimental.pallas import tpu_sc as plsc

NC = 2
NS = 16
NW = NC * NS
NG = 64
K = 80

_MESH = plsc.VectorSubcoreMesh(
    core_axis_name="c", subcore_axis_name="s", num_cores=NC, num_subcores=NS
)


def _npad(N):
  return NS * (-(-N // NS // 8) * 8)


def _build_deg(N, E):
  NP = _npad(N)
  EPW = E // NW
  CH = EPW // K
  RPT = NP // NS

  @functools.partial(
      pl.kernel,
      out_type=jax.ShapeDtypeStruct((NC, NP, 16), jnp.float32),
      mesh=_MESH,
      scratch_types=[
          pltpu.VMEM((K,), jnp.int32),
          pltpu.VMEM((K, 16), jnp.float32),
          pltpu.VMEM((RPT, 16), jnp.float32),
          pltpu.VMEM_SHARED((NP, 16), jnp.float32),
      ],
      compiler_params=pltpu.CompilerParams(use_tc_tiling_on_sc=False),
  )
  def deg_kernel(dst_hbm, degp_hbm, idx_v, ones_v, zbuf_v, table_sh):
    cid = lax.axis_index("c")
    sid = lax.axis_index("s")
    w = cid * NS + sid

    def fill_o(i, c):
      ones_v[i, :] = jnp.ones((16,), jnp.float32)
      return c

    lax.fori_loop(0, K, fill_o, 0)

    def fill_z(i, c):
      zbuf_v[i, :] = jnp.zeros((16,), jnp.float32)
      return c

    lax.fori_loop(0, RPT, fill_z, 0)

    row0 = sid * RPT
    pltpu.sync_copy(zbuf_v, table_sh.at[pl.ds(row0, RPT)])
    plsc.subcore_barrier()

    def chunk(j, carry):
      base = w * EPW + j * K
      pltpu.sync_copy(dst_hbm.at[pl.ds(base, K)], idx_v)
      pltpu.sync_copy(ones_v, table_sh.at[idx_v], add=True)
      return carry

    lax.fori_loop(0, CH, chunk, 0)
    plsc.subcore_barrier()

    pltpu.sync_copy(table_sh.at[pl.ds(row0, RPT)], zbuf_v)
    pltpu.sync_copy(zbuf_v, degp_hbm.at[cid, pl.ds(row0, RPT)])

  return deg_kernel


def _build_prop(N, E, H):
  NP = _npad(N)
  EPW = E // NW
  CH = EPW // K
  RPT = NP // NS
  HH = H // 2

  @functools.partial(
      pl.kernel,
      out_type=jax.ShapeDtypeStruct((2, NC, NP, HH), jnp.float32),
      mesh=_MESH,
      scratch_types=[
          pltpu.VMEM((K,), jnp.int32),
          pltpu.VMEM((K,), jnp.int32),
          pltpu.VMEM((K,), jnp.int32),
          pltpu.VMEM((K, HH), jnp.float32),
          pltpu.VMEM((RPT, HH), jnp.float32),
          pltpu.VMEM((RPT, HH), jnp.float32),
          pltpu.VMEM_SHARED((NP, HH), jnp.float32),
          pltpu.SemaphoreType.DMA,
      ],
      compiler_params=pltpu.CompilerParams(use_tc_tiling_on_sc=False),
  )
  def prop_kernel(hs2_hbm, src_hbm, dst_hbm, parts_hbm, sidx_v, didx_v,
                  gidx_v, rows_v, zbuf_v, obuf_v, acc_sh, sem):
    cid = lax.axis_index("c")
    sid = lax.axis_index("s")
    w = cid * NS + sid

    def fill_z(i, c):
      for k in range(HH // 16):
        zbuf_v[i, pl.ds(k * 16, 16)] = jnp.zeros((16,), jnp.float32)
      return c

    lax.fori_loop(0, RPT, fill_z, 0)
    row0 = sid * RPT

    for h in range(2):
      pltpu.sync_copy(zbuf_v, acc_sh.at[pl.ds(row0, RPT)])
      plsc.subcore_barrier()

      def chunk(j, carry):
        base = w * EPW + j * K
        pltpu.sync_copy(src_hbm.at[pl.ds(base, K)], sidx_v)
        pltpu.sync_copy(dst_hbm.at[pl.ds(base, K)], didx_v)
        for q in range(K // 16):
          s16 = sidx_v[pl.ds(q * 16, 16)]
          gidx_v[pl.ds(q * 16, 16)] = s16 * 2 + h
        pltpu.async_copy(hs2_hbm.at[gidx_v], rows_v, sem).wait()
        pltpu.sync_copy(rows_v, acc_sh.at[didx_v], add=True)
        return carry

      lax.fori_loop(0, CH, chunk, 0)
      plsc.subcore_barrier()

      pltpu.sync_copy(acc_sh.at[pl.ds(row0, RPT)], obuf_v)
      pltpu.sync_copy(obuf_v, parts_hbm.at[h, cid, pl.ds(row0, RPT)])

  return prop_kernel


def _build_pool(Np, H):
  RPW = Np // NW
  CR = 80
  NCH = RPW // CR
  F3 = 3 * H

  @functools.partial(
      pl.kernel,
      out_type=jax.ShapeDtypeStruct((NW, NG, F3), jnp.float32),
      mesh=_MESH,
      scratch_types=[
          pltpu.VMEM((CR,), jnp.int32),
          pltpu.VMEM((CR, H), jnp.float32),
          pltpu.VMEM((CR, H), jnp.float32),
          pltpu.VMEM((CR, H), jnp.float32),
          pltpu.VMEM((NG, F3), jnp.float32),
      ],
      compiler_params=pltpu.CompilerParams(use_tc_tiling_on_sc=False),
  )
  def pool_kernel(x1_hbm, x2_hbm, x3_hbm, batch_hbm, neginf_hbm, out_hbm,
                  bidx_v, xa_v, xb_v, xc_v, pbuf_v):
    cid = lax.axis_index("c")
    sid = lax.axis_index("s")
    w = cid * NS + sid

    pltpu.sync_copy(neginf_hbm, pbuf_v)

    def chunkfn(t, carry):
      base = w * RPW + t * CR
      pltpu.sync_copy(batch_hbm.at[pl.ds(base, CR)], bidx_v)
      pltpu.sync_copy(x1_hbm.at[pl.ds(base, CR)], xa_v)
      pltpu.sync_copy(x2_hbm.at[pl.ds(base, CR)], xb_v)
      pltpu.sync_copy(x3_hbm.at[pl.ds(base, CR)], xc_v)

      def grpfn(q, c2):
        gvec = bidx_v[pl.ds(q * 16, 16)]
        for i in range(16):
          g = gvec[i]
          r = q * 16 + i
          for a, buf in enumerate((xa_v, xb_v, xc_v)):
            for k in range(H // 16):
              v = buf[r, pl.ds(k * 16, 16)]
              cur = pbuf_v[g, pl.ds(a * H + k * 16, 16)]
              pbuf_v[g, pl.ds(a * H + k * 16, 16)] = jnp.maximum(cur, v)
        return c2

      lax.fori_loop(0, CR // 16, grpfn, 0)
      return carry

    lax.fori_loop(0, NCH, chunkfn, 0)
    pltpu.sync_copy(pbuf_v, out_hbm.at[w])

  return pool_kernel


def _tc_first(degp, x, W1):
  N, Fin = x.shape
  H = W1.shape[1]
  B = 1000

  def body(degp_ref, x_ref, w_ref, dinv_ref, hs_ref):
    deg = 1.0 + degp_ref[0, :, 0:1] + degp_ref[1, :, 0:1]
    dinv = 1.0 / jnp.sqrt(deg)
    dinv_ref[...] = dinv
    hs_ref[...] = jnp.dot(
        x_ref[...], w_ref[...], preferred_element_type=jnp.float32) * dinv

  return pl.pallas_call(
      body,
      grid=(N // B,),
      in_specs=[
          pl.BlockSpec((NC, B, 16), lambda i: (0, i, 0)),
          pl.BlockSpec((B, Fin), lambda i: (i, 0)),
          pl.BlockSpec((Fin, H), lambda i: (0, 0)),
      ],
      out_specs=[
          pl.BlockSpec((B, 1), lambda i: (i, 0)),
          pl.BlockSpec((B, H), lambda i: (i, 0)),
      ],
      out_shape=[
          jax.ShapeDtypeStruct((N, 1), jnp.float32),
          jax.ShapeDtypeStruct((N, H), jnp.float32),
      ],
  )(degp, x, W1)


def _tc_combine(parts, hs, dinv, b_row, Wn):
  N, H = hs.shape
  B = 1000

  def body(p_ref, hs_ref, dinv_ref, b_ref, w_ref, y_ref, hsn_ref):
    psum = jnp.concatenate(
        [p_ref[0, 0] + p_ref[0, 1], p_ref[1, 0] + p_ref[1, 1]], axis=1)
    y = dinv_ref[...] * (psum + hs_ref[...]) + b_ref[...]
    y = jnp.maximum(y, 0.0)
    y_ref[...] = y
    hsn_ref[...] = jnp.dot(
        y, w_ref[...], preferred_element_type=jnp.float32) * dinv_ref[...]

  return pl.pallas_call(
      body,
      grid=(N // B,),
      in_specs=[
          pl.BlockSpec((2, NC, B, H // 2), lambda i: (0, 0, i, 0)),
          pl.BlockSpec((B, H), lambda i: (i, 0)),
          pl.BlockSpec((B, 1), lambda i: (i, 0)),
          pl.BlockSpec((1, H), lambda i: (0, 0)),
          pl.BlockSpec((H, H), lambda i: (0, 0)),
      ],
      out_specs=[
          pl.BlockSpec((B, H), lambda i: (i, 0)),
          pl.BlockSpec((B, H), lambda i: (i, 0)),
      ],
      out_shape=[
          jax.ShapeDtypeStruct((N, H), jnp.float32),
          jax.ShapeDtypeStruct((N, H), jnp.float32),
      ],
  )(parts, hs, dinv, b_row, Wn)


def _tc_combine_last(parts, hs, dinv, b_row):
  N, H = hs.shape
  B = 1000

  def body(p_ref, hs_ref, dinv_ref, b_ref, y_ref):
    psum = jnp.concatenate(
        [p_ref[0, 0] + p_ref[0, 1], p_ref[1, 0] + p_ref[1, 1]], axis=1)
    y = dinv_ref[...] * (psum + hs_ref[...]) + b_ref[...]
    y_ref[...] = jnp.maximum(y, 0.0)

  return pl.pallas_call(
      body,
      grid=(N // B,),
      in_specs=[
          pl.BlockSpec((2, NC, B, H // 2), lambda i: (0, 0, i, 0)),
          pl.BlockSpec((B, H), lambda i: (i, 0)),
          pl.BlockSpec((B, 1), lambda i: (i, 0)),
          pl.BlockSpec((1, H), lambda i: (0, 0)),
      ],
      out_specs=pl.BlockSpec((B, H), lambda i: (i, 0)),
      out_shape=jax.ShapeDtypeStruct((N, H), jnp.float32),
  )(parts, hs, dinv, b_row)


def _tc_head(pp, Wl1, bl1_row, Wl2, bl2_row):
  NGg = pp.shape[1]
  C = Wl2.shape[1]

  def body(pp_ref, w1_ref, b1_ref, w2_ref, b2_ref, o_ref):
    pooled = jnp.max(pp_ref[...], axis=0)
    h = jnp.dot(pooled, w1_ref[...], preferred_element_type=jnp.float32)
    h = jnp.maximum(h + b1_ref[...], 0.0)
    logits = jnp.dot(h, w2_ref[...], preferred_element_type=jnp.float32)
    logits = logits + b2_ref[...]
    m = jnp.max(logits, axis=-1, keepdims=True)
    lse = m + jnp.log(jnp.sum(jnp.exp(logits - m), axis=-1, keepdims=True))
    o_ref[...] = logits - lse

  return pl.pallas_call(
      body,
      out_shape=jax.ShapeDtypeStruct((NGg, C), jnp.float32),
  )(pp, Wl1, bl1_row, Wl2, bl2_row)


def kernel(x, edge_index, batch, W1, b1, W2, b2, W3, b3, Wl1, bl1, Wl2, bl2):
  N, _ = x.shape
  H = W1.shape[1]
  E = edge_index.shape[1]
  C = Wl2.shape[1]
  src = edge_index[0]
  dst = edge_index[1]

  deg_k = _build_deg(N, E)
  prop_k = _build_prop(N, E, H)

  degp = deg_k(dst)
  dinv, hs1 = _tc_first(degp, x, W1)

  bs = jnp.stack([b1.reshape(1, H), b2.reshape(1, H), b3.reshape(1, H)])
  Wn_all = jnp.stack([W2, W3, W3])

  def layer(l, carry):
    hs, ys = carry
    p = prop_k(hs.reshape(2 * N, H // 2), src, dst)
    b_row = lax.dynamic_index_in_dim(bs, l, 0, keepdims=False)
    Wn = lax.dynamic_index_in_dim(Wn_all, l, 0, keepdims=False)
    y, hs_next = _tc_combine(p, hs, dinv, b_row, Wn)
    ys = lax.dynamic_update_slice_in_dim(ys, y[None], l, axis=0)
    return (hs_next, ys)

  ys0 = jnp.zeros((3, N, H), jnp.float32)
  _, ys = lax.fori_loop(0, 3, layer, (hs1, ys0))
  x1, x2, x3 = ys[0], ys[1], ys[2]

  RPW = -(-N // (NW * 80)) * 80
  Np = NW * RPW
  if Np != N:
    pad = jnp.full((Np - N, H), -jnp.inf, jnp.float32)
    x1p = jnp.concatenate([x1, pad], axis=0)
    x2p = jnp.concatenate([x2, pad], axis=0)
    x3p = jnp.concatenate([x3, pad], axis=0)
    batch_p = jnp.concatenate(
        [batch, jnp.full((Np - N,), NG - 1, batch.dtype)])
  else:
    x1p, x2p, x3p, batch_p = x1, x2, x3, batch

  pool_k = _build_pool(Np, H)
  neginf = jnp.full((NG, 3 * H), -jnp.inf, jnp.float32)
  pp = pool_k(x1p, x2p, x3p, batch_p, neginf)

  return _tc_head(pp, Wl1, bl1.reshape(1, H), Wl2, bl2.reshape(1, C))


# speedup vs baseline: 19.0356x; 2.7771x over previous
import functools

import jax
import jax.numpy as jnp
from jax import lax
from jax.experimental import pallas as pl
from jax.experimental.pallas import tpu as pltpu
from jax.experimental.pallas import tpu_sc as plsc

NC = 2
NS = 16
NW = NC * NS
NG = 64
K = 80

_MESH = plsc.VectorSubcoreMesh(
    core_axis_name="c", subcore_axis_name="s", num_cores=NC, num_subcores=NS
)


def _npad(N):
  return NS * (-(-N // NS // 8) * 8)


def _build_deg(N, E):
  NP = _npad(N)
  EPW = E // NW
  CH = EPW // K
  RPT = NP // NS

  @functools.partial(
      pl.kernel,
      out_type=jax.ShapeDtypeStruct((NC, NP, 16), jnp.float32),
      mesh=_MESH,
      scratch_types=[
          pltpu.VMEM((K,), jnp.int32),
          pltpu.VMEM((K, 16), jnp.float32),
          pltpu.VMEM((RPT, 16), jnp.float32),
          pltpu.VMEM_SHARED((NP, 16), jnp.float32),
      ],
      compiler_params=pltpu.CompilerParams(use_tc_tiling_on_sc=False),
  )
  def deg_kernel(dst_hbm, degp_hbm, idx_v, ones_v, zbuf_v, table_sh):
    cid = lax.axis_index("c")
    sid = lax.axis_index("s")
    w = cid * NS + sid

    def fill_o(i, c):
      ones_v[i, :] = jnp.ones((16,), jnp.float32)
      return c

    lax.fori_loop(0, K, fill_o, 0)

    def fill_z(i, c):
      zbuf_v[i, :] = jnp.zeros((16,), jnp.float32)
      return c

    lax.fori_loop(0, RPT, fill_z, 0)

    row0 = sid * RPT
    pltpu.sync_copy(zbuf_v, table_sh.at[pl.ds(row0, RPT)])
    plsc.subcore_barrier()

    def chunk(j, carry):
      base = w * EPW + j * K
      pltpu.sync_copy(dst_hbm.at[pl.ds(base, K)], idx_v)
      pltpu.sync_copy(ones_v, table_sh.at[idx_v], add=True)
      return carry

    lax.fori_loop(0, CH, chunk, 0)
    plsc.subcore_barrier()

    pltpu.sync_copy(table_sh.at[pl.ds(row0, RPT)], zbuf_v)
    pltpu.sync_copy(zbuf_v, degp_hbm.at[cid, pl.ds(row0, RPT)])

  return deg_kernel


def _build_prop(N, E, H):
  NP = _npad(N)
  EPW = E // NW
  CH = EPW // K
  RPT = NP // NS
  HH = H // 2
  G = 5
  NGRP = CH // G
  assert CH % G == 0

  @functools.partial(
      pl.kernel,
      out_type=jax.ShapeDtypeStruct((2, NC, NP, HH), jnp.float32),
      mesh=_MESH,
      scratch_types=[
          pltpu.VMEM((CH, K), jnp.int32),
          pltpu.VMEM((CH, K), jnp.int32),
          pltpu.VMEM((2 * G * K, HH), jnp.float32),
          pltpu.VMEM_SHARED((NP, HH), jnp.float32),
          pltpu.SemaphoreType.DMA((2,)),
          pltpu.SemaphoreType.DMA((2,)),
      ],
      compiler_params=pltpu.CompilerParams(use_tc_tiling_on_sc=False),
  )
  def prop_kernel(hs2_hbm, src3_hbm, dst3_hbm, parts_hbm, gidx_v, didx_v,
                  rows_v, acc_sh, gsem, ssem):
    assert 2 * G * K >= RPT
    cid = lax.axis_index("c")
    sid = lax.axis_index("s")
    w = cid * NS + sid
    row0 = sid * RPT

    def fill_z(i, c):
      for k in range(HH // 16):
        rows_v[i, pl.ds(k * 16, 16)] = jnp.zeros((16,), jnp.float32)
      return c

    def fire_gathers(g, p):
      for i in range(G):
        pltpu.async_copy(
            hs2_hbm.at[gidx_v.at[g * G + i]],
            rows_v.at[pl.ds((p * G + i) * K, K)], gsem.at[p])

    def drain(sem_slot):
      pltpu.make_async_copy(
          hs2_hbm.at[pl.ds(0, G * K)], rows_v.at[pl.ds(0, G * K)],
          sem_slot).wait()

    for h in range(2):
      lax.fori_loop(0, RPT, fill_z, 0)
      pltpu.sync_copy(rows_v.at[pl.ds(0, RPT)], acc_sh.at[pl.ds(row0, RPT)])

      pltpu.sync_copy(src3_hbm.at[w], gidx_v)
      pltpu.sync_copy(dst3_hbm.at[w], didx_v)

      def xform(j, c):
        for q in range(K // 16):
          s16 = gidx_v[j, pl.ds(q * 16, 16)]
          gidx_v[j, pl.ds(q * 16, 16)] = s16 * 2 + h
        return c

      lax.fori_loop(0, CH, xform, 0)
      plsc.subcore_barrier()

      fire_gathers(0, 0)

      def group(g, c):
        p = g % 2
        q = 1 - p

        @pl.when(g >= 1)
        def _():
          drain(ssem.at[q])

        @pl.when(g + 1 < NGRP)
        def _():
          fire_gathers(g + 1, q)

        drain(gsem.at[p])
        for i in range(G):
          pltpu.async_copy(
              rows_v.at[pl.ds((p * G + i) * K, K)],
              acc_sh.at[didx_v.at[g * G + i]], ssem.at[p], add=True)
        return c

      lax.fori_loop(0, NGRP, group, 0)
      drain(ssem.at[(NGRP - 1) % 2])
      plsc.subcore_barrier()

      pltpu.sync_copy(acc_sh.at[pl.ds(row0, RPT)], rows_v.at[pl.ds(0, RPT)])
      pltpu.sync_copy(rows_v.at[pl.ds(0, RPT)], parts_hbm.at[h, cid, pl.ds(row0, RPT)])

  return prop_kernel


def _build_pool(Np, H):
  RPW = Np // NW
  CR = 80
  NCH = RPW // CR
  F3 = 3 * H

  @functools.partial(
      pl.kernel,
      out_type=jax.ShapeDtypeStruct((NW, NG, F3), jnp.float32),
      mesh=_MESH,
      scratch_types=[
          pltpu.VMEM((CR,), jnp.int32),
          pltpu.VMEM((CR, H), jnp.float32),
          pltpu.VMEM((CR, H), jnp.float32),
          pltpu.VMEM((CR, H), jnp.float32),
          pltpu.VMEM((NG, F3), jnp.float32),
      ],
      compiler_params=pltpu.CompilerParams(use_tc_tiling_on_sc=False),
  )
  def pool_kernel(x1_hbm, x2_hbm, x3_hbm, batch_hbm, neginf_hbm, out_hbm,
                  bidx_v, xa_v, xb_v, xc_v, pbuf_v):
    cid = lax.axis_index("c")
    sid = lax.axis_index("s")
    w = cid * NS + sid

    pltpu.sync_copy(neginf_hbm, pbuf_v)

    def chunkfn(t, carry):
      base = w * RPW + t * CR
      pltpu.sync_copy(batch_hbm.at[pl.ds(base, CR)], bidx_v)
      pltpu.sync_copy(x1_hbm.at[pl.ds(base, CR)], xa_v)
      pltpu.sync_copy(x2_hbm.at[pl.ds(base, CR)], xb_v)
      pltpu.sync_copy(x3_hbm.at[pl.ds(base, CR)], xc_v)

      def grpfn(q, c2):
        gvec = bidx_v[pl.ds(q * 16, 16)]
        for i in range(16):
          g = gvec[i]
          r = q * 16 + i
          for a, buf in enumerate((xa_v, xb_v, xc_v)):
            for k in range(H // 16):
              v = buf[r, pl.ds(k * 16, 16)]
              cur = pbuf_v[g, pl.ds(a * H + k * 16, 16)]
              pbuf_v[g, pl.ds(a * H + k * 16, 16)] = jnp.maximum(cur, v)
        return c2

      lax.fori_loop(0, CR // 16, grpfn, 0)
      return carry

    lax.fori_loop(0, NCH, chunkfn, 0)
    pltpu.sync_copy(pbuf_v, out_hbm.at[w])

  return pool_kernel


def _tc_first(degp, x, W1):
  N, Fin = x.shape
  H = W1.shape[1]
  B = 1000

  def body(degp_ref, x_ref, w_ref, dinv_ref, hs_ref):
    deg = 1.0 + degp_ref[0, :, 0:1] + degp_ref[1, :, 0:1]
    dinv = 1.0 / jnp.sqrt(deg)
    dinv_ref[...] = dinv
    hs_ref[...] = jnp.dot(
        x_ref[...], w_ref[...], preferred_element_type=jnp.float32) * dinv

  return pl.pallas_call(
      body,
      grid=(N // B,),
      in_specs=[
          pl.BlockSpec((NC, B, 16), lambda i: (0, i, 0)),
          pl.BlockSpec((B, Fin), lambda i: (i, 0)),
          pl.BlockSpec((Fin, H), lambda i: (0, 0)),
      ],
      out_specs=[
          pl.BlockSpec((B, 1), lambda i: (i, 0)),
          pl.BlockSpec((B, H), lambda i: (i, 0)),
      ],
      out_shape=[
          jax.ShapeDtypeStruct((N, 1), jnp.float32),
          jax.ShapeDtypeStruct((N, H), jnp.float32),
      ],
  )(degp, x, W1)


def _tc_combine(parts, hs, dinv, b_row, Wn):
  N, H = hs.shape
  B = 1000

  def body(p_ref, hs_ref, dinv_ref, b_ref, w_ref, y_ref, hsn_ref):
    psum = jnp.concatenate(
        [p_ref[0, 0] + p_ref[0, 1], p_ref[1, 0] + p_ref[1, 1]], axis=1)
    y = dinv_ref[...] * (psum + hs_ref[...]) + b_ref[...]
    y = jnp.maximum(y, 0.0)
    y_ref[...] = y
    hsn_ref[...] = jnp.dot(
        y, w_ref[...], preferred_element_type=jnp.float32) * dinv_ref[...]

  return pl.pallas_call(
      body,
      grid=(N // B,),
      in_specs=[
          pl.BlockSpec((2, NC, B, H // 2), lambda i: (0, 0, i, 0)),
          pl.BlockSpec((B, H), lambda i: (i, 0)),
          pl.BlockSpec((B, 1), lambda i: (i, 0)),
          pl.BlockSpec((1, H), lambda i: (0, 0)),
          pl.BlockSpec((H, H), lambda i: (0, 0)),
      ],
      out_specs=[
          pl.BlockSpec((B, H), lambda i: (i, 0)),
          pl.BlockSpec((B, H), lambda i: (i, 0)),
      ],
      out_shape=[
          jax.ShapeDtypeStruct((N, H), jnp.float32),
          jax.ShapeDtypeStruct((N, H), jnp.float32),
      ],
  )(parts, hs, dinv, b_row, Wn)


def _tc_combine_last(parts, hs, dinv, b_row):
  N, H = hs.shape
  B = 1000

  def body(p_ref, hs_ref, dinv_ref, b_ref, y_ref):
    psum = jnp.concatenate(
        [p_ref[0, 0] + p_ref[0, 1], p_ref[1, 0] + p_ref[1, 1]], axis=1)
    y = dinv_ref[...] * (psum + hs_ref[...]) + b_ref[...]
    y_ref[...] = jnp.maximum(y, 0.0)

  return pl.pallas_call(
      body,
      grid=(N // B,),
      in_specs=[
          pl.BlockSpec((2, NC, B, H // 2), lambda i: (0, 0, i, 0)),
          pl.BlockSpec((B, H), lambda i: (i, 0)),
          pl.BlockSpec((B, 1), lambda i: (i, 0)),
          pl.BlockSpec((1, H), lambda i: (0, 0)),
      ],
      out_specs=pl.BlockSpec((B, H), lambda i: (i, 0)),
      out_shape=jax.ShapeDtypeStruct((N, H), jnp.float32),
  )(parts, hs, dinv, b_row)


def _tc_head(pp, Wl1, bl1_row, Wl2, bl2_row):
  NGg = pp.shape[1]
  C = Wl2.shape[1]

  def body(pp_ref, w1_ref, b1_ref, w2_ref, b2_ref, o_ref):
    pooled = jnp.max(pp_ref[...], axis=0)
    h = jnp.dot(pooled, w1_ref[...], preferred_element_type=jnp.float32)
    h = jnp.maximum(h + b1_ref[...], 0.0)
    logits = jnp.dot(h, w2_ref[...], preferred_element_type=jnp.float32)
    logits = logits + b2_ref[...]
    m = jnp.max(logits, axis=-1, keepdims=True)
    lse = m + jnp.log(jnp.sum(jnp.exp(logits - m), axis=-1, keepdims=True))
    o_ref[...] = logits - lse

  return pl.pallas_call(
      body,
      out_shape=jax.ShapeDtypeStruct((NGg, C), jnp.float32),
  )(pp, Wl1, bl1_row, Wl2, bl2_row)


def kernel(x, edge_index, batch, W1, b1, W2, b2, W3, b3, Wl1, bl1, Wl2, bl2):
  N, _ = x.shape
  H = W1.shape[1]
  E = edge_index.shape[1]
  C = Wl2.shape[1]
  src = edge_index[0]
  dst = edge_index[1]

  deg_k = _build_deg(N, E)
  prop_k = _build_prop(N, E, H)

  degp = deg_k(dst)
  dinv, hs1 = _tc_first(degp, x, W1)

  bs = jnp.stack([b1.reshape(1, H), b2.reshape(1, H), b3.reshape(1, H)])
  Wn_all = jnp.stack([W2, W3, W3])

  EPW = E // NW
  CH = EPW // K
  src3 = src.reshape(NW, CH, K)
  dst3 = dst.reshape(NW, CH, K)

  def layer(l, carry):
    hs, ys = carry
    p = prop_k(hs.reshape(2 * N, H // 2), src3, dst3)
    b_row = lax.dynamic_index_in_dim(bs, l, 0, keepdims=False)
    Wn = lax.dynamic_index_in_dim(Wn_all, l, 0, keepdims=False)
    y, hs_next = _tc_combine(p, hs, dinv, b_row, Wn)
    ys = lax.dynamic_update_slice_in_dim(ys, y[None], l, axis=0)
    return (hs_next, ys)

  ys0 = jnp.zeros((3, N, H), jnp.float32)
  _, ys = lax.fori_loop(0, 3, layer, (hs1, ys0))
  x1, x2, x3 = ys[0], ys[1], ys[2]

  RPW = -(-N // (NW * 80)) * 80
  Np = NW * RPW
  if Np != N:
    pad = jnp.full((Np - N, H), -jnp.inf, jnp.float32)
    x1p = jnp.concatenate([x1, pad], axis=0)
    x2p = jnp.concatenate([x2, pad], axis=0)
    x3p = jnp.concatenate([x3, pad], axis=0)
    batch_p = jnp.concatenate(
        [batch, jnp.full((Np - N,), NG - 1, batch.dtype)])
  else:
    x1p, x2p, x3p, batch_p = x1, x2, x3, batch

  pool_k = _build_pool(Np, H)
  neginf = jnp.full((NG, 3 * H), -jnp.inf, jnp.float32)
  pp = pool_k(x1p, x2p, x3p, batch_p, neginf)

  return _tc_head(pp, Wl1, bl1.reshape(1, H), Wl2, bl2.reshape(1, C))


# speedup vs baseline: 21.2893x; 1.1184x over previous
import functools

import jax
import jax.numpy as jnp
from jax import lax
from jax.experimental import pallas as pl
from jax.experimental.pallas import tpu as pltpu
from jax.experimental.pallas import tpu_sc as plsc

NC = 2
NS = 16
NW = NC * NS
NG = 64
K = 80

_MESH = plsc.VectorSubcoreMesh(
    core_axis_name="c", subcore_axis_name="s", num_cores=NC, num_subcores=NS
)


def _npad(N):
  return NS * (-(-N // NS // 8) * 8)


def _build_deg(N, E):
  NP = _npad(N)
  EPW = E // NW
  CH = EPW // K
  RPT = NP // NS

  @functools.partial(
      pl.kernel,
      out_type=jax.ShapeDtypeStruct((NC, NP, 16), jnp.float32),
      mesh=_MESH,
      scratch_types=[
          pltpu.VMEM((CH, K), jnp.int32),
          pltpu.VMEM((K, 16), jnp.float32),
          pltpu.VMEM((RPT, 16), jnp.float32),
          pltpu.VMEM_SHARED((NP, 16), jnp.float32),
          pltpu.SemaphoreType.DMA,
      ],
      compiler_params=pltpu.CompilerParams(use_tc_tiling_on_sc=False),
  )
  def deg_kernel(dst3_hbm, degp_hbm, didx_v, ones_v, zbuf_v, table_sh, sem):
    cid = lax.axis_index("c")
    sid = lax.axis_index("s")
    w = cid * NS + sid

    def fill_o(i, c):
      ones_v[i, :] = jnp.ones((16,), jnp.float32)
      return c

    lax.fori_loop(0, K, fill_o, 0)

    def fill_z(i, c):
      zbuf_v[i, :] = jnp.zeros((16,), jnp.float32)
      return c

    lax.fori_loop(0, RPT, fill_z, 0)

    row0 = sid * RPT
    pltpu.sync_copy(dst3_hbm.at[w], didx_v)
    pltpu.sync_copy(zbuf_v, table_sh.at[pl.ds(row0, RPT)])
    plsc.subcore_barrier()

    def fire(j, carry):
      pltpu.async_copy(ones_v, table_sh.at[didx_v.at[j]], sem, add=True)
      return carry

    lax.fori_loop(0, CH, fire, 0)

    def drainj(j, carry):
      pltpu.make_async_copy(
          degp_hbm.at[cid, pl.ds(0, K)], ones_v, sem).wait()
      return carry

    lax.fori_loop(0, CH, drainj, 0)
    plsc.subcore_barrier()

    pltpu.sync_copy(table_sh.at[pl.ds(row0, RPT)], zbuf_v)
    pltpu.sync_copy(zbuf_v, degp_hbm.at[cid, pl.ds(row0, RPT)])

  return deg_kernel


def _build_prop(N, E, H):
  NP = _npad(N)
  EPW = E // NW
  CH = EPW // K
  RPT = NP // NS
  HH = H // 2
  G = 5
  NGRP = CH // G
  assert CH % G == 0

  @functools.partial(
      pl.kernel,
      out_type=jax.ShapeDtypeStruct((2, NC, NP, HH), jnp.float32),
      mesh=_MESH,
      scratch_types=[
          pltpu.VMEM((CH, K), jnp.int32),
          pltpu.VMEM((CH, K), jnp.int32),
          pltpu.VMEM((2 * G * K, HH), jnp.float32),
          pltpu.VMEM_SHARED((NP, HH), jnp.float32),
          pltpu.SemaphoreType.DMA((2,)),
          pltpu.SemaphoreType.DMA((2,)),
      ],
      compiler_params=pltpu.CompilerParams(use_tc_tiling_on_sc=False),
  )
  def prop_kernel(hs2_hbm, src3_hbm, dst3_hbm, parts_hbm, gidx_v, didx_v,
                  rows_v, acc_sh, gsem, ssem):
    assert 2 * G * K >= RPT
    cid = lax.axis_index("c")
    sid = lax.axis_index("s")
    w = cid * NS + sid
    row0 = sid * RPT

    def fill_z(i, c):
      for k in range(HH // 16):
        rows_v[i, pl.ds(k * 16, 16)] = jnp.zeros((16,), jnp.float32)
      return c

    def fire_gathers(g, p):
      for i in range(G):
        pltpu.async_copy(
            hs2_hbm.at[gidx_v.at[g * G + i]],
            rows_v.at[pl.ds((p * G + i) * K, K)], gsem.at[p])

    def drain(sem_slot):
      pltpu.make_async_copy(
          hs2_hbm.at[pl.ds(0, G * K)], rows_v.at[pl.ds(0, G * K)],
          sem_slot).wait()

    for h in range(2):
      lax.fori_loop(0, RPT, fill_z, 0)
      pltpu.sync_copy(rows_v.at[pl.ds(0, RPT)], acc_sh.at[pl.ds(row0, RPT)])

      pltpu.sync_copy(src3_hbm.at[w], gidx_v)
      pltpu.sync_copy(dst3_hbm.at[w], didx_v)

      def xform(j, c):
        for q in range(K // 16):
          s16 = gidx_v[j, pl.ds(q * 16, 16)]
          gidx_v[j, pl.ds(q * 16, 16)] = s16 * 2 + h
        return c

      lax.fori_loop(0, CH, xform, 0)
      plsc.subcore_barrier()

      fire_gathers(0, 0)

      def group(g, c):
        p = g % 2
        q = 1 - p

        @pl.when(g >= 1)
        def _():
          drain(ssem.at[q])

        @pl.when(g + 1 < NGRP)
        def _():
          fire_gathers(g + 1, q)

        drain(gsem.at[p])
        for i in range(G):
          pltpu.async_copy(
              rows_v.at[pl.ds((p * G + i) * K, K)],
              acc_sh.at[didx_v.at[g * G + i]], ssem.at[p], add=True)
        return c

      lax.fori_loop(0, NGRP, group, 0)
      drain(ssem.at[(NGRP - 1) % 2])
      plsc.subcore_barrier()

      pltpu.sync_copy(acc_sh.at[pl.ds(row0, RPT)], rows_v.at[pl.ds(0, RPT)])
      pltpu.sync_copy(rows_v.at[pl.ds(0, RPT)], parts_hbm.at[h, cid, pl.ds(row0, RPT)])

  return prop_kernel


def _build_pool(Np, H):
  RPW = Np // NW
  CR = 80
  NCH = RPW // CR
  F3 = 3 * H

  @functools.partial(
      pl.kernel,
      out_type=jax.ShapeDtypeStruct((NW, NG, F3), jnp.float32),
      mesh=_MESH,
      scratch_types=[
          pltpu.VMEM((RPW,), jnp.int32),
          pltpu.VMEM((2 * CR, H), jnp.float32),
          pltpu.VMEM((2 * CR, H), jnp.float32),
          pltpu.VMEM((2 * CR, H), jnp.float32),
          pltpu.VMEM((NG, F3), jnp.float32),
          pltpu.SemaphoreType.DMA((2,)),
      ],
      compiler_params=pltpu.CompilerParams(use_tc_tiling_on_sc=False),
  )
  def pool_kernel(x1_hbm, x2_hbm, x3_hbm, batch_hbm, neginf_hbm, out_hbm,
                  bidx_v, xa_v, xb_v, xc_v, pbuf_v, sem):
    cid = lax.axis_index("c")
    sid = lax.axis_index("s")
    w = cid * NS + sid

    def fire(t, p):
      base = w * RPW + t * CR
      pltpu.async_copy(x1_hbm.at[pl.ds(base, CR)],
                       xa_v.at[pl.ds(p * CR, CR)], sem.at[p])
      pltpu.async_copy(x2_hbm.at[pl.ds(base, CR)],
                       xb_v.at[pl.ds(p * CR, CR)], sem.at[p])
      pltpu.async_copy(x3_hbm.at[pl.ds(base, CR)],
                       xc_v.at[pl.ds(p * CR, CR)], sem.at[p])

    def drain(p):
      for _ in range(3):
        pltpu.make_async_copy(
            x1_hbm.at[pl.ds(0, CR)], xa_v.at[pl.ds(0, CR)], sem.at[p]).wait()

    pltpu.sync_copy(batch_hbm.at[pl.ds(w * RPW, RPW)], bidx_v)
    pltpu.sync_copy(neginf_hbm, pbuf_v)
    fire(0, 0)

    def chunkfn(t, carry):
      p = t % 2

      @pl.when(t + 1 < NCH)
      def _():
        fire(t + 1, 1 - p)

      drain(p)

      def grpfn(q, c2):
        gvec = bidx_v[pl.ds(t * CR + q * 16, 16)]
        for i in range(16):
          g = gvec[i]
          r = p * CR + q * 16 + i
          for a, buf in enumerate((xa_v, xb_v, xc_v)):
            for k in range(H // 16):
              v = buf[r, pl.ds(k * 16, 16)]
              cur = pbuf_v[g, pl.ds(a * H + k * 16, 16)]
              pbuf_v[g, pl.ds(a * H + k * 16, 16)] = jnp.maximum(cur, v)
        return c2

      lax.fori_loop(0, CR // 16, grpfn, 0)
      return carry

    lax.fori_loop(0, NCH, chunkfn, 0)
    pltpu.sync_copy(pbuf_v, out_hbm.at[w])

  return pool_kernel


def _tc_first(degp, x, W1):
  N, Fin = x.shape
  H = W1.shape[1]
  B = 1000

  def body(degp_ref, x_ref, w_ref, dinv_ref, hs_ref):
    deg = 1.0 + degp_ref[0, :, 0:1] + degp_ref[1, :, 0:1]
    dinv = 1.0 / jnp.sqrt(deg)
    dinv_ref[...] = dinv
    hs_ref[...] = jnp.dot(
        x_ref[...], w_ref[...], preferred_element_type=jnp.float32) * dinv

  return pl.pallas_call(
      body,
      grid=(N // B,),
      in_specs=[
          pl.BlockSpec((NC, B, 16), lambda i: (0, i, 0)),
          pl.BlockSpec((B, Fin), lambda i: (i, 0)),
          pl.BlockSpec((Fin, H), lambda i: (0, 0)),
      ],
      out_specs=[
          pl.BlockSpec((B, 1), lambda i: (i, 0)),
          pl.BlockSpec((B, H), lambda i: (i, 0)),
      ],
      out_shape=[
          jax.ShapeDtypeStruct((N, 1), jnp.float32),
          jax.ShapeDtypeStruct((N, H), jnp.float32),
      ],
  )(degp, x, W1)


def _tc_combine(parts, hs, dinv, b_row, Wn):
  N, H = hs.shape
  B = 1000

  def body(p_ref, hs_ref, dinv_ref, b_ref, w_ref, y_ref, hsn_ref):
    psum = jnp.concatenate(
        [p_ref[0, 0] + p_ref[0, 1], p_ref[1, 0] + p_ref[1, 1]], axis=1)
    y = dinv_ref[...] * (psum + hs_ref[...]) + b_ref[...]
    y = jnp.maximum(y, 0.0)
    y_ref[...] = y
    hsn_ref[...] = jnp.dot(
        y, w_ref[...], preferred_element_type=jnp.float32) * dinv_ref[...]

  return pl.pallas_call(
      body,
      grid=(N // B,),
      in_specs=[
          pl.BlockSpec((2, NC, B, H // 2), lambda i: (0, 0, i, 0)),
          pl.BlockSpec((B, H), lambda i: (i, 0)),
          pl.BlockSpec((B, 1), lambda i: (i, 0)),
          pl.BlockSpec((1, H), lambda i: (0, 0)),
          pl.BlockSpec((H, H), lambda i: (0, 0)),
      ],
      out_specs=[
          pl.BlockSpec((B, H), lambda i: (i, 0)),
          pl.BlockSpec((B, H), lambda i: (i, 0)),
      ],
      out_shape=[
          jax.ShapeDtypeStruct((N, H), jnp.float32),
          jax.ShapeDtypeStruct((N, H), jnp.float32),
      ],
  )(parts, hs, dinv, b_row, Wn)


def _tc_combine_last(parts, hs, dinv, b_row):
  N, H = hs.shape
  B = 1000

  def body(p_ref, hs_ref, dinv_ref, b_ref, y_ref):
    psum = jnp.concatenate(
        [p_ref[0, 0] + p_ref[0, 1], p_ref[1, 0] + p_ref[1, 1]], axis=1)
    y = dinv_ref[...] * (psum + hs_ref[...]) + b_ref[...]
    y_ref[...] = jnp.maximum(y, 0.0)

  return pl.pallas_call(
      body,
      grid=(N // B,),
      in_specs=[
          pl.BlockSpec((2, NC, B, H // 2), lambda i: (0, 0, i, 0)),
          pl.BlockSpec((B, H), lambda i: (i, 0)),
          pl.BlockSpec((B, 1), lambda i: (i, 0)),
          pl.BlockSpec((1, H), lambda i: (0, 0)),
      ],
      out_specs=pl.BlockSpec((B, H), lambda i: (i, 0)),
      out_shape=jax.ShapeDtypeStruct((N, H), jnp.float32),
  )(parts, hs, dinv, b_row)


def _tc_head(pp, Wl1, bl1_row, Wl2, bl2_row):
  NGg = pp.shape[1]
  C = Wl2.shape[1]

  def body(pp_ref, w1_ref, b1_ref, w2_ref, b2_ref, o_ref):
    pooled = jnp.max(pp_ref[...], axis=0)
    h = jnp.dot(pooled, w1_ref[...], preferred_element_type=jnp.float32)
    h = jnp.maximum(h + b1_ref[...], 0.0)
    logits = jnp.dot(h, w2_ref[...], preferred_element_type=jnp.float32)
    logits = logits + b2_ref[...]
    m = jnp.max(logits, axis=-1, keepdims=True)
    lse = m + jnp.log(jnp.sum(jnp.exp(logits - m), axis=-1, keepdims=True))
    o_ref[...] = logits - lse

  return pl.pallas_call(
      body,
      out_shape=jax.ShapeDtypeStruct((NGg, C), jnp.float32),
  )(pp, Wl1, bl1_row, Wl2, bl2_row)


def kernel(x, edge_index, batch, W1, b1, W2, b2, W3, b3, Wl1, bl1, Wl2, bl2):
  N, _ = x.shape
  H = W1.shape[1]
  E = edge_index.shape[1]
  C = Wl2.shape[1]
  src = edge_index[0]
  dst = edge_index[1]

  deg_k = _build_deg(N, E)
  prop_k = _build_prop(N, E, H)

  EPW0 = E // NW
  CH0 = EPW0 // K
  src3 = src.reshape(NW, CH0, K)
  dst3 = dst.reshape(NW, CH0, K)

  degp = deg_k(dst3)
  dinv, hs1 = _tc_first(degp, x, W1)

  bs = jnp.stack([b1.reshape(1, H), b2.reshape(1, H), b3.reshape(1, H)])
  Wn_all = jnp.stack([W2, W3, W3])

  def layer(l, carry):
    hs, ys = carry
    p = prop_k(hs.reshape(2 * N, H // 2), src3, dst3)
    b_row = lax.dynamic_index_in_dim(bs, l, 0, keepdims=False)
    Wn = lax.dynamic_index_in_dim(Wn_all, l, 0, keepdims=False)
    y, hs_next = _tc_combine(p, hs, dinv, b_row, Wn)
    ys = lax.dynamic_update_slice_in_dim(ys, y[None], l, axis=0)
    return (hs_next, ys)

  ys0 = jnp.zeros((3, N, H), jnp.float32)
  _, ys = lax.fori_loop(0, 3, layer, (hs1, ys0))
  x1, x2, x3 = ys[0], ys[1], ys[2]

  RPW = -(-N // (NW * 80)) * 80
  Np = NW * RPW
  if Np != N:
    pad = jnp.full((Np - N, H), -jnp.inf, jnp.float32)
    x1p = jnp.concatenate([x1, pad], axis=0)
    x2p = jnp.concatenate([x2, pad], axis=0)
    x3p = jnp.concatenate([x3, pad], axis=0)
    batch_p = jnp.concatenate(
        [batch, jnp.full((Np - N,), NG - 1, batch.dtype)])
  else:
    x1p, x2p, x3p, batch_p = x1, x2, x3, batch

  pool_k = _build_pool(Np, H)
  neginf = jnp.full((NG, 3 * H), -jnp.inf, jnp.float32)
  pp = pool_k(x1p, x2p, x3p, batch_p, neginf)

  return _tc_head(pp, Wl1, bl1.reshape(1, H), Wl2, bl2.reshape(1, C))
